# baseline (speedup 1.0000x reference)
import functools

import jax
import jax.numpy as jnp
from jax import lax
from jax.experimental import pallas as pl
from jax.experimental.pallas import tpu as pltpu
from jax.experimental.pallas import tpu_sc as plsc

L = 16
NS = 16
N_IDS = 4 * 8192
CHUNK = N_IDS // NS
D = 768
SEG = D // 3
BLK = 2 * L

_mesh = plsc.VectorSubcoreMesh(core_axis_name="c", subcore_axis_name="s",
                               num_cores=1)


def _lane_sum(vec, scratch_ref):
    lanes = jnp.arange(L, dtype=jnp.int32)
    for stride in (1, 2, 4, 8):
        scratch_ref[...] = vec
        vec = vec + plsc.load_gather(scratch_ref, [lanes ^ stride])
    return vec


def _sc_embed_sum_body(ids_hbm, table_hbm, out_hbm, comb_hbm,
                       ids_v, part_v, blk_v, all_v, tab_v, fv_v,
                       sem_i, sem_t):
    sid = lax.axis_index("s")

    cp_ids = pltpu.async_copy(ids_hbm.at[pl.ds(sid * CHUNK, CHUNK)],
                              ids_v, sem_i)

    for k in range(6):
        @pl.when(sid == 10 + k)
        def _table_seg(k=k):
            pltpu.async_copy(table_hbm.at[pl.ds(k * SEG, SEG)], tab_v,
                             sem_t).wait()
            r_a = tab_v[pl.ds(0, L)]
            r_b = tab_v[pl.ds(L, L)]
            for j in range(2, SEG // L, 2):
                r_a = r_a + tab_v[pl.ds(j * L, L)]
                r_b = r_b + tab_v[pl.ds((j + 1) * L, L)]
            rs = _lane_sum(r_a + r_b, fv_v)
            blk_v[pl.ds(L, L)] = plsc.bitcast(rs, jnp.int32)

    cp_ids.wait()
    a0 = ids_v[pl.ds(0, L)]
    a1 = ids_v[pl.ds(L, L)]
    for i in range(2, CHUNK // L, 2):
        a0 = a0 + ids_v[pl.ds(i * L, L)]
        a1 = a1 + ids_v[pl.ds((i + 1) * L, L)]
    blk_v[pl.ds(0, L)] = _lane_sum(a0 + a1, part_v)
    pltpu.sync_copy(blk_v, comb_hbm.at[pl.ds(sid * BLK, BLK)])

    plsc.subcore_barrier()

    @pl.when(sid == 0)
    def _finalize():
        pltpu.sync_copy(comb_hbm, all_v)
        t0 = all_v[pl.ds(0, L)]
        t1 = all_v[pl.ds(BLK, L)]
        for i in range(2, NS, 2):
            t0 = t0 + all_v[pl.ds(i * BLK, L)]
            t1 = t1 + all_v[pl.ds((i + 1) * BLK, L)]
        n1 = (t0 + t1).astype(jnp.float32)

        def seg(k):
            return plsc.bitcast(all_v[pl.ds((10 + k) * BLK + L, L)],
                                jnp.float32)

        r0s = seg(0) + seg(1) + seg(2)
        r1s = seg(3) + seg(4) + seg(5)
        res = (jnp.float32(N_IDS) - n1) * r0s + n1 * r1s
        fv_v[...] = res
        pltpu.sync_copy(fv_v, out_hbm)


_sc_embed_sum = functools.partial(
    pl.kernel,
    mesh=_mesh,
    out_type=(jax.ShapeDtypeStruct((L,), jnp.float32),
              jax.ShapeDtypeStruct((NS * BLK,), jnp.int32)),
    compiler_params=pltpu.CompilerParams(needs_layout_passes=False),
    scratch_types=[
        pltpu.VMEM((CHUNK,), jnp.int32),
        pltpu.VMEM((L,), jnp.int32),
        pltpu.VMEM((BLK,), jnp.int32),
        pltpu.VMEM((NS * BLK,), jnp.int32),
        pltpu.VMEM((SEG,), jnp.float32),
        pltpu.VMEM((L,), jnp.float32),
        pltpu.SemaphoreType.DMA,
        pltpu.SemaphoreType.DMA,
    ],
)(_sc_embed_sum_body)


def kernel(token_type_ids, table):
    ids = token_type_ids.reshape(-1).astype(jnp.int32)
    res, _ = _sc_embed_sum(ids, table.reshape(-1))
    return res[0]


# speedup vs baseline: 1.0113x; 1.0113x over previous
import functools

import jax
import jax.numpy as jnp
from jax import lax
from jax.experimental import pallas as pl
from jax.experimental.pallas import tpu as pltpu
from jax.experimental.pallas import tpu_sc as plsc

L = 16
NS = 16
N_IDS = 4 * 8192
CHUNK = N_IDS // NS
D = 768
SEG = D // 3
NP = NS + 6

_mesh = plsc.VectorSubcoreMesh(core_axis_name="c", subcore_axis_name="s",
                               num_cores=1)


def _lane_sum(vec, scratch_ref):
    lanes = jnp.arange(L, dtype=jnp.int32)
    for stride in (1, 2, 4, 8):
        scratch_ref[...] = vec
        vec = vec + plsc.load_gather(scratch_ref, [lanes ^ stride])
    return vec


def _sc_embed_sum_body(ids_hbm, table_hbm, out_hbm, comb_hbm,
                       ids_v, part_v, all_v, tab_v, fv_v, sem_i, sem_t):
    sid = lax.axis_index("s")

    cp_ids = pltpu.async_copy(ids_hbm.at[pl.ds(sid * CHUNK, CHUNK)],
                              ids_v, sem_i)

    for k in range(6):
        @pl.when(sid == 10 + k)
        def _table_seg(k=k):
            pltpu.async_copy(table_hbm.at[pl.ds(k * SEG, SEG)], tab_v,
                             sem_t).wait()
            r_a = tab_v[pl.ds(0, L)]
            r_b = tab_v[pl.ds(L, L)]
            for j in range(2, SEG // L, 2):
                r_a = r_a + tab_v[pl.ds(j * L, L)]
                r_b = r_b + tab_v[pl.ds((j + 1) * L, L)]
            rs = _lane_sum(r_a + r_b, fv_v)
            part_v[...] = plsc.bitcast(rs, jnp.int32)
            pltpu.sync_copy(part_v, comb_hbm.at[pl.ds((NS + k) * L, L)])

    cp_ids.wait()
    a0 = ids_v[pl.ds(0, L)]
    a1 = ids_v[pl.ds(L, L)]
    for i in range(2, CHUNK // L, 2):
        a0 = a0 + ids_v[pl.ds(i * L, L)]
        a1 = a1 + ids_v[pl.ds((i + 1) * L, L)]
    part_v[...] = _lane_sum(a0 + a1, part_v)
    pltpu.sync_copy(part_v, comb_hbm.at[pl.ds(sid * L, L)])

    plsc.subcore_barrier()

    @pl.when(sid == 0)
    def _finalize():
        pltpu.sync_copy(comb_hbm, all_v)
        t0 = all_v[pl.ds(0, L)]
        t1 = all_v[pl.ds(L, L)]
        for i in range(2, NS, 2):
            t0 = t0 + all_v[pl.ds(i * L, L)]
            t1 = t1 + all_v[pl.ds((i + 1) * L, L)]
        n1 = (t0 + t1).astype(jnp.float32)

        def seg(k):
            return plsc.bitcast(all_v[pl.ds((NS + k) * L, L)], jnp.float32)

        r0s = seg(0) + seg(1) + seg(2)
        r1s = seg(3) + seg(4) + seg(5)
        res = (jnp.float32(N_IDS) - n1) * r0s + n1 * r1s
        fv_v[...] = res
        pltpu.sync_copy(fv_v, out_hbm)


_sc_embed_sum = functools.partial(
    pl.kernel,
    mesh=_mesh,
    out_type=(jax.ShapeDtypeStruct((L,), jnp.float32),
              jax.ShapeDtypeStruct((NP * L,), jnp.int32)),
    compiler_params=pltpu.CompilerParams(needs_layout_passes=False),
    scratch_types=[
        pltpu.VMEM((CHUNK,), jnp.int32),
        pltpu.VMEM((L,), jnp.int32),
        pltpu.VMEM((NP * L,), jnp.int32),
        pltpu.VMEM((SEG,), jnp.float32),
        pltpu.VMEM((L,), jnp.float32),
        pltpu.SemaphoreType.DMA,
        pltpu.SemaphoreType.DMA,
    ],
)(_sc_embed_sum_body)


def kernel(token_type_ids, table):
    ids = token_type_ids.reshape(-1).astype(jnp.int32)
    res, _ = _sc_embed_sum(ids, table.reshape(-1))
    return res[0]


# speedup vs baseline: 1.0179x; 1.0065x over previous
import functools

import jax
import jax.numpy as jnp
from jax import lax
from jax.experimental import pallas as pl
from jax.experimental.pallas import tpu as pltpu
from jax.experimental.pallas import tpu_sc as plsc

L = 16
NS = 16
N_IDS = 4 * 8192
CHUNK = N_IDS // NS
D = 768
SEG = D // 3
NP = NS + 6

_mesh = plsc.VectorSubcoreMesh(core_axis_name="c", subcore_axis_name="s",
                               num_cores=1)


def _lane_sum(vec, scratch_ref):
    lanes = jnp.arange(L, dtype=jnp.int32)
    for stride in (1, 2, 4, 8):
        scratch_ref[...] = vec
        vec = vec + plsc.load_gather(scratch_ref, [lanes ^ stride])
    return vec


def _sc_embed_sum_body(ids_hbm, table_hbm, out_hbm, comb_hbm,
                       ids_v, part_v, seg_v, all_v, tab_v, fv_v,
                       sem_i, sem_t):
    sid = lax.axis_index("s")

    cp_ids = pltpu.async_copy(ids_hbm.at[pl.ds(sid * CHUNK, CHUNK)],
                              ids_v, sem_i)

    for k in range(6):
        @pl.when(sid == 10 + k)
        def _table_seg(k=k):
            pltpu.async_copy(table_hbm.at[pl.ds(k * SEG, SEG)], tab_v,
                             sem_t).wait()
            r_a = tab_v[pl.ds(0, L)]
            r_b = tab_v[pl.ds(L, L)]
            for j in range(2, SEG // L, 2):
                r_a = r_a + tab_v[pl.ds(j * L, L)]
                r_b = r_b + tab_v[pl.ds((j + 1) * L, L)]
            rs = _lane_sum(r_a + r_b, fv_v)
            seg_v[...] = plsc.bitcast(rs, jnp.int32)
            pltpu.async_copy(seg_v, comb_hbm.at[pl.ds((NS + k) * L, L)],
                             sem_t)

    cp_ids.wait()
    a0 = ids_v[pl.ds(0, L)]
    a1 = ids_v[pl.ds(L, L)]
    for i in range(2, CHUNK // L, 2):
        a0 = a0 + ids_v[pl.ds(i * L, L)]
        a1 = a1 + ids_v[pl.ds((i + 1) * L, L)]
    part_v[...] = _lane_sum(a0 + a1, part_v)
    pltpu.sync_copy(part_v, comb_hbm.at[pl.ds(sid * L, L)])

    for k in range(6):
        @pl.when(sid == 10 + k)
        def _drain_seg(k=k):
            pltpu.make_async_copy(
                seg_v, comb_hbm.at[pl.ds((NS + k) * L, L)], sem_t).wait()

    plsc.subcore_barrier()

    @pl.when(sid == 0)
    def _finalize():
        pltpu.sync_copy(comb_hbm, all_v)
        t0 = all_v[pl.ds(0, L)]
        t1 = all_v[pl.ds(L, L)]
        for i in range(2, NS, 2):
            t0 = t0 + all_v[pl.ds(i * L, L)]
            t1 = t1 + all_v[pl.ds((i + 1) * L, L)]
        n1 = (t0 + t1).astype(jnp.float32)

        def seg(k):
            return plsc.bitcast(all_v[pl.ds((NS + k) * L, L)], jnp.float32)

        r0s = seg(0) + seg(1) + seg(2)
        r1s = seg(3) + seg(4) + seg(5)
        res = (jnp.float32(N_IDS) - n1) * r0s + n1 * r1s
        fv_v[...] = res
        pltpu.sync_copy(fv_v, out_hbm)


_sc_embed_sum = functools.partial(
    pl.kernel,
    mesh=_mesh,
    out_type=(jax.ShapeDtypeStruct((L,), jnp.float32),
              jax.ShapeDtypeStruct((NP * L,), jnp.int32)),
    compiler_params=pltpu.CompilerParams(needs_layout_passes=False),
    scratch_types=[
        pltpu.VMEM((CHUNK,), jnp.int32),
        pltpu.VMEM((L,), jnp.int32),
        pltpu.VMEM((L,), jnp.int32),
        pltpu.VMEM((NP * L,), jnp.int32),
        pltpu.VMEM((SEG,), jnp.float32),
        pltpu.VMEM((L,), jnp.float32),
        pltpu.SemaphoreType.DMA,
        pltpu.SemaphoreType.DMA,
    ],
)(_sc_embed_sum_body)


def kernel(token_type_ids, table):
    ids = token_type_ids.reshape(-1).astype(jnp.int32)
    res, _ = _sc_embed_sum(ids, table.reshape(-1))
    return res[0]


# speedup vs baseline: 1.0410x; 1.0227x over previous
import functools

import jax
import jax.numpy as jnp
from jax import lax
from jax.experimental import pallas as pl
from jax.experimental.pallas import tpu as pltpu
from jax.experimental.pallas import tpu_sc as plsc

L = 16
NS = 16
N_IDS = 4 * 8192
CHUNK = N_IDS // NS
D = 768
SEG = D // 3
NP = NS + 6

_mesh = plsc.VectorSubcoreMesh(core_axis_name="c", subcore_axis_name="s",
                               num_cores=1)


def _lane_sum(vec, scratch_ref):
    lanes = jnp.arange(L, dtype=jnp.int32)
    for stride in (1, 2, 4, 8):
        scratch_ref[...] = vec
        vec = vec + plsc.load_gather(scratch_ref, [lanes ^ stride])
    return vec


def _sc_embed_sum_body(ids_hbm, table_hbm, out_hbm,
                       ids_v, part_v, seg_v, all_v, tab_v, fv_v, shared,
                       sem_i, sem_t):
    sid = lax.axis_index("s")

    cp_ids = pltpu.async_copy(ids_hbm.at[pl.ds(sid * CHUNK, CHUNK)],
                              ids_v, sem_i)

    for k in range(6):
        @pl.when(sid == 10 + k)
        def _table_seg(k=k):
            pltpu.async_copy(table_hbm.at[pl.ds(k * SEG, SEG)], tab_v,
                             sem_t).wait()
            r_a = tab_v[pl.ds(0, L)]
            r_b = tab_v[pl.ds(L, L)]
            for j in range(2, SEG // L, 2):
                r_a = r_a + tab_v[pl.ds(j * L, L)]
                r_b = r_b + tab_v[pl.ds((j + 1) * L, L)]
            rs = _lane_sum(r_a + r_b, fv_v)
            seg_v[...] = plsc.bitcast(rs, jnp.int32)
            pltpu.async_copy(seg_v, shared.at[pl.ds((NS + k) * L, L)],
                             sem_t)

    cp_ids.wait()
    a0 = ids_v[pl.ds(0, L)]
    a1 = ids_v[pl.ds(L, L)]
    for i in range(2, CHUNK // L, 2):
        a0 = a0 + ids_v[pl.ds(i * L, L)]
        a1 = a1 + ids_v[pl.ds((i + 1) * L, L)]
    part_v[...] = _lane_sum(a0 + a1, part_v)
    pltpu.sync_copy(part_v, shared.at[pl.ds(sid * L, L)])

    for k in range(6):
        @pl.when(sid == 10 + k)
        def _drain_seg(k=k):
            pltpu.make_async_copy(
                seg_v, shared.at[pl.ds((NS + k) * L, L)], sem_t).wait()

    plsc.subcore_barrier()

    @pl.when(sid == 0)
    def _finalize():
        pltpu.sync_copy(shared, all_v)
        t0 = all_v[pl.ds(0, L)]
        t1 = all_v[pl.ds(L, L)]
        for i in range(2, NS, 2):
            t0 = t0 + all_v[pl.ds(i * L, L)]
            t1 = t1 + all_v[pl.ds((i + 1) * L, L)]
        n1 = (t0 + t1).astype(jnp.float32)

        def seg(k):
            return plsc.bitcast(all_v[pl.ds((NS + k) * L, L)], jnp.float32)

        r0s = seg(0) + seg(1) + seg(2)
        r1s = seg(3) + seg(4) + seg(5)
        res = (jnp.float32(N_IDS) - n1) * r0s + n1 * r1s
        fv_v[...] = res
        pltpu.sync_copy(fv_v, out_hbm)


_sc_embed_sum = functools.partial(
    pl.kernel,
    mesh=_mesh,
    out_type=jax.ShapeDtypeStruct((L,), jnp.float32),
    compiler_params=pltpu.CompilerParams(needs_layout_passes=False),
    scratch_types=[
        pltpu.VMEM((CHUNK,), jnp.int32),
        pltpu.VMEM((L,), jnp.int32),
        pltpu.VMEM((L,), jnp.int32),
        pltpu.VMEM((NP * L,), jnp.int32),
        pltpu.VMEM((SEG,), jnp.float32),
        pltpu.VMEM((L,), jnp.float32),
        pltpu.VMEM_SHARED((NP * L,), jnp.int32),
        pltpu.SemaphoreType.DMA,
        pltpu.SemaphoreType.DMA,
    ],
)(_sc_embed_sum_body)


def kernel(token_type_ids, table):
    ids = token_type_ids.reshape(-1).astype(jnp.int32)
    res = _sc_embed_sum(ids, table.reshape(-1))
    return res[0]
